# Initial kernel scaffold; baseline (speedup 1.0000x reference)
import jax
import jax.numpy as jnp
from jax import lax
from jax.experimental import pallas as pl
from jax.experimental.pallas import tpu as pltpu
from jax.experimental.pallas import tpu_sc as plsc

N = 10000
E = 320000
D = 128
NG = 64
OUT_DIM = 10

NC = 2
NS = 16
NW = NC * NS
EB = 128
NBLK = 79
EPT = NBLK * EB
E_PAD = NW * EPT
N_PAD = 10240
RPT = N_PAD // NS


def _mm_body(x_ref, w_ref, o_ref):
    o_ref[...] = jnp.dot(x_ref[...], w_ref[...],
                         preferred_element_type=jnp.float32)


def _tc_matmul(x, w):
    return pl.pallas_call(
        _mm_body,
        out_shape=jax.ShapeDtypeStruct((x.shape[0], w.shape[1]), jnp.float32),
    )(x, w)


def _bn_relu(p0, p1, b, g, be):
    agg = p0 + p1 + b
    mean = jnp.mean(agg, axis=0, keepdims=True)
    var = jnp.mean(agg * agg, axis=0, keepdims=True) - mean * mean
    normed = (agg - mean) * lax.rsqrt(var + 1e-5) * g + be
    return jnp.maximum(normed, 0.0)


def _bn_mm_body(p0_ref, p1_ref, b_ref, g_ref, be_ref, w_ref, o_ref):
    r = _bn_relu(p0_ref[...], p1_ref[...], b_ref[...], g_ref[...], be_ref[...])
    o_ref[...] = jnp.dot(r, w_ref[...], preferred_element_type=jnp.float32)


def _tc_bn_matmul(p0, p1, b, g, be, w):
    return pl.pallas_call(
        _bn_mm_body,
        out_shape=jax.ShapeDtypeStruct((N, D), jnp.float32),
    )(p0, p1, b, g, be, w)


def _final_body(p0_ref, p1_ref, b_ref, g_ref, be_ref, batch_ref, fcw_ref,
                fcb_ref, o_ref):
    r = _bn_relu(p0_ref[...], p1_ref[...], b_ref[...], g_ref[...], be_ref[...])
    ids = lax.broadcasted_iota(jnp.int32, (NG, N), 0)
    oh = (ids == batch_ref[...]).astype(jnp.float32)
    cnt = jnp.sum(oh, axis=1, keepdims=True)
    ohs = oh / jnp.maximum(cnt, 1.0)
    pooled = jnp.dot(ohs, r, preferred_element_type=jnp.float32)
    o_ref[...] = (jnp.dot(pooled, fcw_ref[...],
                          preferred_element_type=jnp.float32) + fcb_ref[...])


def _tc_final(p0, p1, b, g, be, batch2d, fcw_pad, fcb_pad):
    return pl.pallas_call(
        _final_body,
        out_shape=jax.ShapeDtypeStruct((NG, D), jnp.float32),
    )(p0, p1, b, g, be, batch2d, fcw_pad, fcb_pad)


def _sc_layer_body(src_hbm, dst_hbm, h_hbm, px_hbm, py_hbm, pz_hbm, par_hbm,
                   out_hbm,
                   src_v, dst_v, qs_v, qd_v, kern_v, rows_v,
                   pxc_v, pyc_v, pzc_v, qdc_v, qsc_v, par_v,
                   agg_sh, q_sh, sem):
    cid = lax.axis_index("c")
    sid = lax.axis_index("s")
    tid = cid * NS + sid
    r0 = sid * RPT

    pltpu.sync_copy(par_hbm, par_v)
    kx = par_v[0]
    ky = par_v[1]
    kz = par_v[2]
    kb = par_v[3]

    pltpu.sync_copy(px_hbm.at[pl.ds(r0, RPT)], pxc_v)
    pltpu.sync_copy(py_hbm.at[pl.ds(r0, RPT)], pyc_v)
    pltpu.sync_copy(pz_hbm.at[pl.ds(r0, RPT)], pzc_v)

    @pl.loop(0, RPT, step=16)
    def _q_loop(i):
        qd = (pxc_v[pl.ds(i, 16)] * kx + pyc_v[pl.ds(i, 16)] * ky
              + pzc_v[pl.ds(i, 16)] * kz)
        qdc_v[pl.ds(i, 16)] = qd
        qsc_v[pl.ds(i, 16)] = qd - kb

    pltpu.sync_copy(qdc_v, q_sh.at[0, pl.ds(r0, RPT)])
    pltpu.sync_copy(qsc_v, q_sh.at[1, pl.ds(r0, RPT)])

    @pl.loop(0, EB)
    def _z_loop(r):
        for c in range(D // 16):
            rows_v[r, pl.ds(16 * c, 16)] = jnp.zeros((16,), jnp.float32)

    for k in range(RPT // EB):
        pltpu.sync_copy(rows_v, agg_sh.at[pl.ds(r0 + EB * k, EB)])

    plsc.subcore_barrier()

    pltpu.sync_copy(q_sh.at[0], qd_v)
    pltpu.sync_copy(q_sh.at[1], qs_v)

    pltpu.sync_copy(src_hbm.at[tid], src_v)
    pltpu.sync_copy(dst_hbm.at[tid], dst_v)

    @pl.loop(0, NBLK)
    def _kern_loop(j):
        for i in range(EB // 16):
            s_idx = src_v[j, pl.ds(16 * i, 16)]
            d_idx = dst_v[j, pl.ds(16 * i, 16)]
            z = plsc.load_gather(qd_v, [d_idx]) - plsc.load_gather(qs_v, [s_idx])
            e = jnp.exp(z + z)
            kern_v[j, pl.ds(16 * i, 16)] = 1.0 - 2.0 / (e + 1.0)

    @pl.loop(0, NBLK)
    def _edge_loop(j):
        pltpu.async_copy(h_hbm.at[src_v.at[j]], rows_v, sem).wait()

        @pl.loop(0, EB)
        def _scale_loop(r):
            kv = kern_v[j, r]
            for c in range(D // 16):
                sl = (r, pl.ds(16 * c, 16))
                rows_v[sl] = rows_v[sl] * kv

        pltpu.sync_copy(rows_v, agg_sh.at[dst_v.at[j]], add=True)

    plsc.subcore_barrier()

    pltpu.sync_copy(agg_sh.at[pl.ds(r0, RPT)],
                    out_hbm.at[cid].at[pl.ds(r0, RPT)])


def _sc_layer(src3, dst3, h, px, py, pz, par):
    mesh = plsc.VectorSubcoreMesh(core_axis_name="c", subcore_axis_name="s")
    f32 = jnp.float32
    kern = pl.kernel(
        _sc_layer_body,
        out_type=jax.ShapeDtypeStruct((NC, N_PAD, D), f32),
        mesh=mesh,
        scratch_types=[
            pltpu.VMEM((NBLK, EB), jnp.int32),
            pltpu.VMEM((NBLK, EB), jnp.int32),
            pltpu.VMEM((N_PAD,), f32),
            pltpu.VMEM((N_PAD,), f32),
            pltpu.VMEM((NBLK, EB), f32),
            pltpu.VMEM((EB, D), f32),
            pltpu.VMEM((RPT,), f32),
            pltpu.VMEM((RPT,), f32),
            pltpu.VMEM((RPT,), f32),
            pltpu.VMEM((RPT,), f32),
            pltpu.VMEM((RPT,), f32),
            pltpu.VMEM((16,), f32),
            pltpu.VMEM_SHARED((N_PAD, D), f32),
            pltpu.VMEM_SHARED((2, N_PAD), f32),
            pltpu.SemaphoreType.DMA,
        ],
    )
    return kern(src3, dst3, h, px, py, pz, par)


def kernel(x, pos, edge_index, batch, W0, b0, kW0, kb0, g0, be0,
           W1, b1, kW1, kb1, g1, be1, W2, b2, kW2, kb2, g2, be2, fcW, fcb):
    f32 = jnp.float32

    src = edge_index[0]
    dst = edge_index[1]
    npad = E_PAD - E
    src3 = jnp.concatenate([src, jnp.zeros((npad,), jnp.int32)]
                           ).reshape(NW, NBLK, EB)
    dst3 = jnp.concatenate([dst, jnp.full((npad,), N, jnp.int32)]
                           ).reshape(NW, NBLK, EB)

    zpad = jnp.zeros((N_PAD - N,), f32)
    px = jnp.concatenate([pos[:, 0], zpad])
    py = jnp.concatenate([pos[:, 1], zpad])
    pz = jnp.concatenate([pos[:, 2], zpad])

    def params_vec(kW, kb):
        return jnp.concatenate([kW.reshape(3), kb.reshape(1),
                                jnp.zeros((12,), f32)])

    par = [params_vec(kW0, kb0), params_vec(kW1, kb1), params_vec(kW2, kb2)]
    bs = [b.reshape(1, D) for b in (b0, b1, b2)]
    gs = [g.reshape(1, D) for g in (g0, g1, g2)]
    bes = [be.reshape(1, D) for be in (be0, be1, be2)]

    h = _tc_matmul(x, W0)
    parts = _sc_layer(src3, dst3, h, px, py, pz, par[0])
    p0, p1 = parts[0, :N], parts[1, :N]

    h = _tc_bn_matmul(p0, p1, bs[0], gs[0], bes[0], W1)
    parts = _sc_layer(src3, dst3, h, px, py, pz, par[1])
    p0, p1 = parts[0, :N], parts[1, :N]

    h = _tc_bn_matmul(p0, p1, bs[1], gs[1], bes[1], W2)
    parts = _sc_layer(src3, dst3, h, px, py, pz, par[2])
    p0, p1 = parts[0, :N], parts[1, :N]

    batch2d = batch.reshape(1, N).astype(jnp.int32)
    fcw_pad = jnp.zeros((D, D), f32).at[:, :OUT_DIM].set(fcW)
    fcb_pad = jnp.zeros((1, D), f32).at[0, :OUT_DIM].set(fcb)
    out = _tc_final(p0, p1, bs[2], gs[2], bes[2], batch2d, fcw_pad, fcb_pad)
    return out[:, :OUT_DIM]


# speedup vs baseline: 4.9913x; 4.9913x over previous
import dataclasses

import jax
import jax.numpy as jnp
from jax import lax
from jax.experimental import pallas as pl
from jax.experimental.pallas import tpu as pltpu
from jax.experimental.pallas import tpu_sc as plsc

N = 10000
E = 320000
D = 128
NG = 64
OUT_DIM = 10

NC = 2
NS = 16
NW = NC * NS
EB = 128
NBLK = 79
EPT = NBLK * EB
E_PAD = NW * EPT
N_PAD = 10240
RPT = N_PAD // NS


def _bn_relu(p0, p1, b, g, be):
    agg = p0 + p1 + b
    mean = jnp.mean(agg, axis=0, keepdims=True)
    var = jnp.mean(agg * agg, axis=0, keepdims=True) - mean * mean
    normed = (agg - mean) * lax.rsqrt(var + 1e-5) * g + be
    return jnp.maximum(normed, 0.0)


def _q_proj(pos8_ref, kw8_ref):
    return jnp.dot(pos8_ref[...], kw8_ref[...],
                   preferred_element_type=jnp.float32)[:, 0:1]


def _mm_body(x_ref, w_ref, pos8_ref, kw8_ref, o_ref, q_ref):
    o_ref[...] = jnp.dot(x_ref[...], w_ref[...],
                         preferred_element_type=jnp.float32)
    q_ref[...] = _q_proj(pos8_ref, kw8_ref)


def _tc_matmul_q(x, w, pos8, kw8):
    return pl.pallas_call(
        _mm_body,
        out_shape=[jax.ShapeDtypeStruct((N, D), jnp.float32),
                   jax.ShapeDtypeStruct((N, 1), jnp.float32)],
    )(x, w, pos8, kw8)


def _bn_mm_body(p0_ref, p1_ref, b_ref, g_ref, be_ref, w_ref, pos8_ref,
                kw8_ref, o_ref, q_ref):
    r = _bn_relu(p0_ref[...], p1_ref[...], b_ref[...], g_ref[...], be_ref[...])
    o_ref[...] = jnp.dot(r, w_ref[...], preferred_element_type=jnp.float32)
    q_ref[...] = _q_proj(pos8_ref, kw8_ref)


def _tc_bn_matmul_q(p0, p1, b, g, be, w, pos8, kw8):
    return pl.pallas_call(
        _bn_mm_body,
        out_shape=[jax.ShapeDtypeStruct((N, D), jnp.float32),
                   jax.ShapeDtypeStruct((N, 1), jnp.float32)],
    )(p0, p1, b, g, be, w, pos8, kw8)


def _final_body(p0_ref, p1_ref, b_ref, g_ref, be_ref, batch_ref, fcw_ref,
                fcb_ref, o_ref):
    r = _bn_relu(p0_ref[...], p1_ref[...], b_ref[...], g_ref[...], be_ref[...])
    ids = lax.broadcasted_iota(jnp.int32, (NG, N), 0)
    oh = (ids == batch_ref[...]).astype(jnp.float32)
    cnt = jnp.sum(oh, axis=1, keepdims=True)
    ohs = oh / jnp.maximum(cnt, 1.0)
    pooled = jnp.dot(ohs, r, preferred_element_type=jnp.float32)
    o_ref[...] = (jnp.dot(pooled, fcw_ref[...],
                          preferred_element_type=jnp.float32) + fcb_ref[...])


def _tc_final(p0, p1, b, g, be, batch2d, fcw_pad, fcb_pad):
    return pl.pallas_call(
        _final_body,
        out_shape=jax.ShapeDtypeStruct((NG, D), jnp.float32),
    )(p0, p1, b, g, be, batch2d, fcw_pad, fcb_pad)


def _sc_layer_body(src_hbm, dst_hbm, h_hbm, q_hbm, par_hbm, out_hbm,
                   src_v, dst_v, q_v, kern_b, rows_v, par_v,
                   agg_sh, sem):
    cid = lax.axis_index("c")
    sid = lax.axis_index("s")
    tid = cid * NS + sid
    r0 = sid * RPT

    pltpu.sync_copy(par_hbm, par_v)
    kb = par_v[pl.ds(0, 16)][0]

    @pl.loop(0, EB)
    def _z_loop(r):
        for c in range(D // 16):
            rows_v[r, pl.ds(16 * c, 16)] = jnp.zeros((16,), jnp.float32)

    for k in range(RPT // EB):
        pltpu.sync_copy(rows_v, agg_sh.at[pl.ds(r0 + EB * k, EB)])

    pltpu.sync_copy(q_hbm, q_v)
    pltpu.sync_copy(src_hbm.at[tid], src_v)
    pltpu.sync_copy(dst_hbm.at[tid], dst_v)

    plsc.subcore_barrier()

    @pl.loop(0, NBLK)
    def _edge_loop(j):
        pltpu.async_copy(h_hbm.at[src_v.at[j]], rows_v, sem).wait()

        for i in range(EB // 16):
            s_idx = src_v[j, pl.ds(16 * i, 16)]
            d_idx = dst_v[j, pl.ds(16 * i, 16)]
            z = (plsc.load_gather(q_v, [d_idx])
                 - plsc.load_gather(q_v, [s_idx]) + kb)
            e = jnp.exp(z + z)
            kern_b[pl.ds(16 * i, 16)] = 1.0 - 2.0 / (e + 1.0)

        @pl.loop(0, EB, step=16)
        def _scale_loop(r):
            kvv = kern_b[pl.ds(r, 16)]
            for i in range(16):
                kv = kvv[i]
                for c in range(D // 16):
                    sl = (r + i, pl.ds(16 * c, 16))
                    rows_v[sl] = rows_v[sl] * kv

        pltpu.sync_copy(rows_v, agg_sh.at[dst_v.at[j]], add=True)

    plsc.subcore_barrier()

    pltpu.sync_copy(agg_sh.at[pl.ds(r0, RPT)],
                    out_hbm.at[cid].at[pl.ds(r0, RPT)])


def _sc_layer(src3, dst3, h, q, par):
    mesh = plsc.VectorSubcoreMesh(core_axis_name="c", subcore_axis_name="s")
    f32 = jnp.float32
    cp = pltpu.CompilerParams()
    if "needs_layout_passes" in pltpu.CompilerParams.__dataclass_fields__:
        cp = dataclasses.replace(cp, needs_layout_passes=False)
    kern = pl.kernel(
        _sc_layer_body,
        out_type=jax.ShapeDtypeStruct((NC, N_PAD, D), f32),
        mesh=mesh,
        compiler_params=cp,
        scratch_types=[
            pltpu.VMEM((NBLK, EB), jnp.int32),
            pltpu.VMEM((NBLK, EB), jnp.int32),
            pltpu.VMEM((N_PAD,), f32),
            pltpu.VMEM((EB,), f32),
            pltpu.VMEM((EB, D), f32),
            pltpu.VMEM((16,), f32),
            pltpu.VMEM_SHARED((N_PAD, D), f32),
            pltpu.SemaphoreType.DMA,
        ],
    )
    return kern(src3, dst3, h, q, par)


def kernel(x, pos, edge_index, batch, W0, b0, kW0, kb0, g0, be0,
           W1, b1, kW1, kb1, g1, be1, W2, b2, kW2, kb2, g2, be2, fcW, fcb):
    f32 = jnp.float32

    src = edge_index[0]
    dst = edge_index[1]
    npad = E_PAD - E
    src3 = jnp.concatenate([src, jnp.zeros((npad,), jnp.int32)]
                           ).reshape(NW, NBLK, EB)
    dst3 = jnp.concatenate([dst, jnp.full((npad,), N, jnp.int32)]
                           ).reshape(NW, NBLK, EB)

    pos8 = jnp.concatenate([pos, jnp.zeros((N, 5), f32)], axis=1)

    def kw_pad(kW):
        return jnp.zeros((8, D), f32).at[:3, 0].set(kW.reshape(3))

    def par_vec(kb):
        return jnp.concatenate([kb.reshape(1), jnp.zeros((15,), f32)])

    def q_pad(q2d):
        return jnp.concatenate([q2d[:, 0], jnp.zeros((N_PAD - N,), f32)])

    kws = [kw_pad(kW0), kw_pad(kW1), kw_pad(kW2)]
    pars = [par_vec(kb0), par_vec(kb1), par_vec(kb2)]
    bs = [b.reshape(1, D) for b in (b0, b1, b2)]
    gs = [g.reshape(1, D) for g in (g0, g1, g2)]
    bes = [be.reshape(1, D) for be in (be0, be1, be2)]

    h, q2d = _tc_matmul_q(x, W0, pos8, kws[0])
    parts = _sc_layer(src3, dst3, h, q_pad(q2d), pars[0])
    p0, p1 = parts[0, :N], parts[1, :N]

    h, q2d = _tc_bn_matmul_q(p0, p1, bs[0], gs[0], bes[0], W1, pos8, kws[1])
    parts = _sc_layer(src3, dst3, h, q_pad(q2d), pars[1])
    p0, p1 = parts[0, :N], parts[1, :N]

    h, q2d = _tc_bn_matmul_q(p0, p1, bs[1], gs[1], bes[1], W2, pos8, kws[2])
    parts = _sc_layer(src3, dst3, h, q_pad(q2d), pars[2])
    p0, p1 = parts[0, :N], parts[1, :N]

    batch2d = batch.reshape(1, N).astype(jnp.int32)
    fcw_pad = jnp.zeros((D, D), f32).at[:, :OUT_DIM].set(fcW)
    fcb_pad = jnp.zeros((1, D), f32).at[0, :OUT_DIM].set(fcb)
    out = _tc_final(p0, p1, bs[2], gs[2], bes[2], batch2d, fcw_pad, fcb_pad)
    return out[:, :OUT_DIM]
